# baseline (speedup 1.0000x reference)
import jax
import jax.numpy as jnp
from jax.experimental import pallas as pl


def _mlp_body(h_ref, w1, b1, w2, b2, w3, b3, o_ref):
    h = h_ref[...]
    h = jax.nn.relu(h @ w1[...].T + b1[...])
    h = jax.nn.relu(h @ w2[...].T + b2[...])
    h = h @ w3[...].T + b3[...]
    o_ref[...] = h


def _softmax_body(h_ref, o_ref):
    h = h_ref[...]
    m = jnp.max(h, axis=1, keepdims=True)
    e = jnp.exp(h - m)
    o_ref[...] = e / jnp.sum(e, axis=1, keepdims=True)


def kernel(x, edge_index, edge_attr, conv_params, lin_params):
    src = edge_index[0]
    dst = edge_index[1]
    n = x.shape[0]
    h = x
    for (Wr, Wn, b) in conv_params:
        msg = jnp.take(h, src, axis=0) * edge_attr[:, None]
        agg = jax.ops.segment_sum(msg, dst, num_segments=n)
        h = h @ Wr.T + agg @ Wn.T + b
        h = jax.nn.relu(h)
    h = h.reshape(-1, 396)
    (w1, b1), (w2, b2), (w3, b3) = lin_params
    h = pl.pallas_call(
        _mlp_body,
        out_shape=jax.ShapeDtypeStruct((505, 396), jnp.float32),
    )(h, w1, b1, w2, b2, w3, b3)
    h = h.reshape(-1, 2)
    nrows = h.shape[0]
    blk = 8192
    grid = (nrows + blk - 1) // blk
    out = pl.pallas_call(
        _softmax_body,
        grid=(grid,),
        in_specs=[pl.BlockSpec((blk, 2), lambda i: (i, 0))],
        out_specs=pl.BlockSpec((blk, 2), lambda i: (i, 0)),
        out_shape=jax.ShapeDtypeStruct((nrows, 2), jnp.float32),
    )(h)
    return out


# speedup vs baseline: 3.0277x; 3.0277x over previous
import functools

import jax
import jax.numpy as jnp
from jax import lax
from jax.experimental import pallas as pl
from jax.experimental.pallas import tpu as pltpu
from jax.experimental.pallas import tpu_sc as plsc

N = 99990
E = 1599840
NC, NS, LANES = 2, 16, 16
NW = NC * NS
B = 128
WINDOWS = 391
E_TILE = B * WINDOWS
E_PAD = E_TILE * NW
N_PAD = 100352
ROWS_PER_TILE = N_PAD // NS
ZB = 64
DOUTS = (50, 50, 50, 50, 50, 20, 15, 10, 5, 2)
BLK = 1024


def _ceil_div(a, b):
    return (a + b - 1) // b


def _make_sc_segsum(G):
    mesh = plsc.VectorSubcoreMesh(
        core_axis_name="c", subcore_axis_name="s", num_cores=NC, num_subcores=NS
    )

    def body(g_hbm, src_hbm, dst_hbm, w_hbm, out_hbm,
             src_v, dst_v, w_v, rows_v, zero_v, acc, sem):
        c = lax.axis_index("c")
        s = lax.axis_index("s")
        tid = c * NS + s
        row0 = s * ROWS_PER_TILE
        ebase = tid * E_TILE

        def zb(j, carry):
            zero_v[j, :] = jnp.zeros((LANES,), jnp.float32)
            return carry

        lax.fori_loop(0, ZB, zb, 0)

        for gi in range(G):
            def zc(j, carry):
                pltpu.sync_copy(zero_v, acc.at[pl.ds(row0 + j * ZB, ZB)])
                return carry

            lax.fori_loop(0, ROWS_PER_TILE // ZB, zc, 0)
            plsc.subcore_barrier()

            def win(wi, carry):
                base = ebase + wi * B
                pltpu.sync_copy(src_hbm.at[pl.ds(base, B)], src_v)
                pltpu.sync_copy(dst_hbm.at[pl.ds(base, B)], dst_v)
                pltpu.sync_copy(w_hbm.at[pl.ds(base, B)], w_v)
                pltpu.async_copy(g_hbm.at[gi].at[src_v], rows_v, sem).wait()

                def mul(g2, inner):
                    wv = w_v[pl.ds(g2 * LANES, LANES)]
                    for j in range(LANES):
                        i = g2 * LANES + j
                        rows_v[i, :] = rows_v[i, :] * jnp.full(
                            (LANES,), wv[j], jnp.float32)
                    return inner

                lax.fori_loop(0, B // LANES, mul, 0)
                pltpu.sync_copy(rows_v, acc.at[dst_v], add=True)
                return carry

            lax.fori_loop(0, WINDOWS, win, 0)
            plsc.subcore_barrier()
            pltpu.sync_copy(
                acc.at[pl.ds(row0, ROWS_PER_TILE)],
                out_hbm.at[c].at[gi].at[pl.ds(row0, ROWS_PER_TILE)])
        return

    return pl.kernel(
        body,
        out_type=jax.ShapeDtypeStruct((NC, G, N_PAD, LANES), jnp.float32),
        mesh=mesh,
        compiler_params=pltpu.CompilerParams(use_tc_tiling_on_sc=False),
        scratch_types=[
            pltpu.VMEM((B,), jnp.int32),
            pltpu.VMEM((B,), jnp.int32),
            pltpu.VMEM((B,), jnp.float32),
            pltpu.VMEM((B, LANES), jnp.float32),
            pltpu.VMEM((ZB, LANES), jnp.float32),
            pltpu.VMEM_SHARED((N_PAD, LANES), jnp.float32),
            pltpu.SemaphoreType.DMA,
        ],
    )


def _pad_cols(a, width):
    d = a.shape[-1]
    if d == width:
        return a
    return jnp.concatenate(
        [a, jnp.zeros(a.shape[:-1] + (width - d,), a.dtype)], axis=-1)


def _agg_cat(agg, gp, d):
    parts = [agg[0, j] + agg[1, j] for j in range(gp)]
    return jnp.concatenate(parts, axis=-1)[:, :d] if gp > 1 else parts[0][:, :d]


def _tc_first_body(x_ref, agg_ref, wr0, wn0, b0, wn1, wr1, b1, g_ref, r_ref):
    a = _agg_cat(agg_ref[...], 1, 5)
    h = jax.nn.relu(
        jnp.dot(x_ref[...], wr0[...].T, preferred_element_type=jnp.float32, precision=lax.Precision.HIGHEST)
        + jnp.dot(a, wn0[...].T, preferred_element_type=jnp.float32, precision=lax.Precision.HIGHEST)
        + b0[...])
    g = jnp.dot(h, wn1[...].T, preferred_element_type=jnp.float32, precision=lax.Precision.HIGHEST)
    gq = _pad_cols(g, g_ref.shape[0] * LANES)
    for j in range(g_ref.shape[0]):
        g_ref[j] = gq[:, j * LANES:(j + 1) * LANES]
    r_ref[...] = jnp.dot(h, wr1[...].T,
                         preferred_element_type=jnp.float32, precision=lax.Precision.HIGHEST) + b1[...]


def _tc_mid_body(r_in_ref, agg_ref, wn, wr, b, g_ref, r_ref, *, gp, dp):
    h = jax.nn.relu(r_in_ref[...] + _agg_cat(agg_ref[...], gp, dp))
    g = jnp.dot(h, wn[...].T, preferred_element_type=jnp.float32, precision=lax.Precision.HIGHEST)
    gq = _pad_cols(g, g_ref.shape[0] * LANES)
    for j in range(g_ref.shape[0]):
        g_ref[j] = gq[:, j * LANES:(j + 1) * LANES]
    r_ref[...] = jnp.dot(h, wr[...].T,
                         preferred_element_type=jnp.float32, precision=lax.Precision.HIGHEST) + b[...]


def _tc_last_body(r_in_ref, agg_ref, o_ref):
    o_ref[...] = jax.nn.relu(r_in_ref[...] + _agg_cat(agg_ref[...], 1, 2))


def _mlp_body(h_ref, w1, b1, w2, b2, w3, b3, o_ref):
    h = h_ref[...]
    h = jax.nn.relu(
        jnp.dot(h, w1[...].T, preferred_element_type=jnp.float32, precision=lax.Precision.HIGHEST) + b1[...])
    h = jax.nn.relu(
        jnp.dot(h, w2[...].T, preferred_element_type=jnp.float32, precision=lax.Precision.HIGHEST) + b2[...])
    h = jnp.dot(h, w3[...].T, preferred_element_type=jnp.float32, precision=lax.Precision.HIGHEST) + b3[...]
    o_ref[...] = h


def _softmax_body(h_ref, o_ref):
    h = h_ref[...]
    m = jnp.max(h, axis=1, keepdims=True)
    e = jnp.exp(h - m)
    o_ref[...] = e / jnp.sum(e, axis=1, keepdims=True)


def _full_spec(shape):
    n = len(shape)
    return pl.BlockSpec(shape, lambda i: (0,) * n)


def kernel(x, edge_index, edge_attr, conv_params, lin_params):
    src = edge_index[0]
    dst = edge_index[1]

    pad = E_PAD - E
    spread = (jnp.arange(pad, dtype=jnp.int32) * 1021) % N
    src_p = jnp.concatenate([src, spread])
    dst_p = jnp.concatenate([dst, spread])
    w_p = jnp.concatenate([edge_attr, jnp.zeros((pad,), jnp.float32)])

    grid = (_ceil_div(N, BLK),)
    gspecs = {}

    def sc_call(g_blocked, G):
        return _make_sc_segsum(G)(g_blocked, src_p, dst_p, w_p)

    g0 = _pad_cols(x, LANES)[None]
    agg0 = sc_call(g0, 1)

    (wr0, wn0, b0) = conv_params[0]
    (wr1w, wn1w, b1w) = conv_params[1]
    G1 = _ceil_div(DOUTS[1], LANES)
    g1, r1 = pl.pallas_call(
        _tc_first_body,
        grid=grid,
        in_specs=[
            pl.BlockSpec((BLK, 5), lambda i: (i, 0)),
            pl.BlockSpec((NC, 1, BLK, LANES), lambda i: (0, 0, i, 0)),
            _full_spec(wr0.shape), _full_spec(wn0.shape), _full_spec(b0.shape),
            _full_spec(wn1w.shape), _full_spec(wr1w.shape), _full_spec(b1w.shape),
        ],
        out_specs=[
            pl.BlockSpec((G1, BLK, LANES), lambda i: (0, i, 0)),
            pl.BlockSpec((BLK, DOUTS[1]), lambda i: (i, 0)),
        ],
        out_shape=[
            jax.ShapeDtypeStruct((G1, N, LANES), jnp.float32),
            jax.ShapeDtypeStruct((N, DOUTS[1]), jnp.float32),
        ],
    )(x, agg0, wr0, wn0, b0, wn1w, wr1w, b1w)

    g_cur, r_cur = g1, r1
    for k in range(1, 9):
        Gk = _ceil_div(DOUTS[k], LANES)
        agg = sc_call(g_cur, Gk)
        dnext = DOUTS[k + 1]
        Gn = _ceil_div(dnext, LANES)
        (wrn, wnn, bn) = conv_params[k + 1]
        body = functools.partial(_tc_mid_body, gp=Gk, dp=DOUTS[k])
        g_cur, r_cur = pl.pallas_call(
            body,
            grid=grid,
            in_specs=[
                pl.BlockSpec((BLK, DOUTS[k]), lambda i: (i, 0)),
                pl.BlockSpec((NC, Gk, BLK, LANES), lambda i: (0, 0, i, 0)),
                _full_spec(wnn.shape), _full_spec(wrn.shape), _full_spec(bn.shape),
            ],
            out_specs=[
                pl.BlockSpec((Gn, BLK, LANES), lambda i: (0, i, 0)),
                pl.BlockSpec((BLK, dnext), lambda i: (i, 0)),
            ],
            out_shape=[
                jax.ShapeDtypeStruct((Gn, N, LANES), jnp.float32),
                jax.ShapeDtypeStruct((N, dnext), jnp.float32),
            ],
        )(r_cur, agg, wnn, wrn, bn)

    agg9 = sc_call(g_cur, 1)
    h10 = pl.pallas_call(
        _tc_last_body,
        grid=grid,
        in_specs=[
            pl.BlockSpec((BLK, 2), lambda i: (i, 0)),
            pl.BlockSpec((NC, 1, BLK, LANES), lambda i: (0, 0, i, 0)),
        ],
        out_specs=pl.BlockSpec((BLK, 2), lambda i: (i, 0)),
        out_shape=jax.ShapeDtypeStruct((N, 2), jnp.float32),
    )(r_cur, agg9)

    h = h10.reshape(-1, 396)
    (w1, b1), (w2, b2), (w3, b3) = lin_params
    h = pl.pallas_call(
        _mlp_body,
        out_shape=jax.ShapeDtypeStruct((505, 396), jnp.float32),
    )(h, w1, b1, w2, b2, w3, b3)
    h = h.reshape(-1, 2)
    sblk = 8192
    out = pl.pallas_call(
        _softmax_body,
        grid=(_ceil_div(N, sblk),),
        in_specs=[pl.BlockSpec((sblk, 2), lambda i: (i, 0))],
        out_specs=pl.BlockSpec((sblk, 2), lambda i: (i, 0)),
        out_shape=jax.ShapeDtypeStruct((N, 2), jnp.float32),
    )(h)
    return out


# speedup vs baseline: 7.2439x; 2.3925x over previous
import functools

import jax
import jax.numpy as jnp
from jax import lax
from jax.experimental import pallas as pl
from jax.experimental.pallas import tpu as pltpu
from jax.experimental.pallas import tpu_sc as plsc

N = 99990
E = 1599840
NC, NS, LANES = 2, 16, 16
NW = NC * NS
B = 128
K = 16
CH = B * K
NCHUNK = 25
E_TILE = CH * NCHUNK
E_PAD = E_TILE * NW
N_PAD = 100352
ROWS_PER_TILE = N_PAD // NS
ZB = 64
DOUTS = (50, 50, 50, 50, 50, 20, 15, 10, 5, 2)
BLK = 1024


def _ceil_div(a, b):
    return (a + b - 1) // b


def _make_sc_segsum(G):
    mesh = plsc.VectorSubcoreMesh(
        core_axis_name="c", subcore_axis_name="s", num_cores=NC, num_subcores=NS
    )

    def body(g_hbm, src_hbm, dst_hbm, w_hbm, out_hbm,
             src_v, dst_v, w_v, rows0, rows1, zero_v, acc,
             isem, gsem0, gsem1, ssem0, ssem1):
        c = lax.axis_index("c")
        s = lax.axis_index("s")
        tid = c * NS + s
        row0 = s * ROWS_PER_TILE
        ebase = tid * E_TILE
        rows = (rows0, rows1)
        gsems = (gsem0, gsem1)
        ssems = (ssem0, ssem1)

        def zb(j, carry):
            zero_v[j, :] = jnp.zeros((LANES,), jnp.float32)
            return carry

        lax.fori_loop(0, ZB, zb, 0)

        for gi in range(G):
            g2d = g_hbm.at[gi]

            def zc(j, carry):
                pltpu.sync_copy(zero_v, acc.at[pl.ds(row0 + j * ZB, ZB)])
                return carry

            lax.fori_loop(0, ROWS_PER_TILE // ZB, zc, 0)
            plsc.subcore_barrier()

            def chunk(ci, carry):
                rbase = (ebase // B) + ci * K
                wbase = ebase + ci * CH
                d1 = pltpu.async_copy(
                    src_hbm.at[pl.ds(rbase, K)], src_v, isem)
                d2 = pltpu.async_copy(
                    dst_hbm.at[pl.ds(rbase, K)], dst_v, isem)
                d3 = pltpu.async_copy(w_hbm.at[pl.ds(wbase, CH)], w_v, isem)
                d1.wait()
                d2.wait()
                d3.wait()
                gd = [None, None]
                sd = [None, None]
                gd[0] = pltpu.async_copy(g2d.at[src_v.at[0]], rows[0], gsems[0])
                for wi in range(K):
                    buf = wi & 1
                    if wi + 1 < K:
                        nb = (wi + 1) & 1
                        if sd[nb] is not None:
                            sd[nb].wait()
                            sd[nb] = None
                        gd[nb] = pltpu.async_copy(
                            g2d.at[src_v.at[wi + 1]], rows[nb], gsems[nb])
                    gd[buf].wait()
                    rbuf = rows[buf]

                    def mul(g2, inner, _wi=wi, _rbuf=rbuf):
                        wv = w_v[pl.ds(_wi * B + g2 * LANES, LANES)]
                        for j in range(LANES):
                            i = g2 * LANES + j
                            _rbuf[i, :] = _rbuf[i, :] * jnp.full(
                                (LANES,), wv[j], jnp.float32)
                        return inner

                    lax.fori_loop(0, B // LANES, mul, 0)
                    sd[buf] = pltpu.async_copy(
                        rbuf, acc.at[dst_v.at[wi]], ssems[buf], add=True)
                for b2 in range(2):
                    if sd[b2] is not None:
                        sd[b2].wait()
                return carry

            lax.fori_loop(0, NCHUNK, chunk, 0)
            plsc.subcore_barrier()
            pltpu.sync_copy(
                acc.at[pl.ds(row0, ROWS_PER_TILE)],
                out_hbm.at[c].at[gi].at[pl.ds(row0, ROWS_PER_TILE)])
        return

    return pl.kernel(
        body,
        out_type=jax.ShapeDtypeStruct((NC, G, N_PAD, LANES), jnp.float32),
        mesh=mesh,
        compiler_params=pltpu.CompilerParams(use_tc_tiling_on_sc=False),
        scratch_types=[
            pltpu.VMEM((K, B), jnp.int32),
            pltpu.VMEM((K, B), jnp.int32),
            pltpu.VMEM((CH,), jnp.float32),
            pltpu.VMEM((B, LANES), jnp.float32),
            pltpu.VMEM((B, LANES), jnp.float32),
            pltpu.VMEM((ZB, LANES), jnp.float32),
            pltpu.VMEM_SHARED((N_PAD, LANES), jnp.float32),
            pltpu.SemaphoreType.DMA,
            pltpu.SemaphoreType.DMA,
            pltpu.SemaphoreType.DMA,
            pltpu.SemaphoreType.DMA,
            pltpu.SemaphoreType.DMA,
        ],
    )


def _pad_cols(a, width):
    d = a.shape[-1]
    if d == width:
        return a
    return jnp.concatenate(
        [a, jnp.zeros(a.shape[:-1] + (width - d,), a.dtype)], axis=-1)


def _agg_cat(agg, gp, d):
    parts = [agg[0, j] + agg[1, j] for j in range(gp)]
    return jnp.concatenate(parts, axis=-1)[:, :d] if gp > 1 else parts[0][:, :d]


def _tc_first_body(x_ref, agg_ref, wr0, wn0, b0, wn1, wr1, b1, g_ref, r_ref):
    a = _agg_cat(agg_ref[...], 1, 5)
    h = jax.nn.relu(
        jnp.dot(x_ref[...], wr0[...].T, preferred_element_type=jnp.float32, precision=lax.Precision.HIGHEST)
        + jnp.dot(a, wn0[...].T, preferred_element_type=jnp.float32, precision=lax.Precision.HIGHEST)
        + b0[...])
    g = jnp.dot(h, wn1[...].T, preferred_element_type=jnp.float32, precision=lax.Precision.HIGHEST)
    gq = _pad_cols(g, g_ref.shape[0] * LANES)
    for j in range(g_ref.shape[0]):
        g_ref[j] = gq[:, j * LANES:(j + 1) * LANES]
    r_ref[...] = jnp.dot(h, wr1[...].T,
                         preferred_element_type=jnp.float32, precision=lax.Precision.HIGHEST) + b1[...]


def _tc_mid_body(r_in_ref, agg_ref, wn, wr, b, g_ref, r_ref, *, gp, dp):
    h = jax.nn.relu(r_in_ref[...] + _agg_cat(agg_ref[...], gp, dp))
    g = jnp.dot(h, wn[...].T, preferred_element_type=jnp.float32, precision=lax.Precision.HIGHEST)
    gq = _pad_cols(g, g_ref.shape[0] * LANES)
    for j in range(g_ref.shape[0]):
        g_ref[j] = gq[:, j * LANES:(j + 1) * LANES]
    r_ref[...] = jnp.dot(h, wr[...].T,
                         preferred_element_type=jnp.float32, precision=lax.Precision.HIGHEST) + b[...]


def _tc_last_body(r_in_ref, agg_ref, o_ref):
    o_ref[...] = jax.nn.relu(r_in_ref[...] + _agg_cat(agg_ref[...], 1, 2))


def _mlp_body(h_ref, w1, b1, w2, b2, w3, b3, o_ref):
    h = h_ref[...]
    h = jax.nn.relu(
        jnp.dot(h, w1[...].T, preferred_element_type=jnp.float32, precision=lax.Precision.HIGHEST) + b1[...])
    h = jax.nn.relu(
        jnp.dot(h, w2[...].T, preferred_element_type=jnp.float32, precision=lax.Precision.HIGHEST) + b2[...])
    h = jnp.dot(h, w3[...].T, preferred_element_type=jnp.float32, precision=lax.Precision.HIGHEST) + b3[...]
    o_ref[...] = h


def _softmax_body(h_ref, o_ref):
    h = h_ref[...]
    m = jnp.max(h, axis=1, keepdims=True)
    e = jnp.exp(h - m)
    o_ref[...] = e / jnp.sum(e, axis=1, keepdims=True)


def _full_spec(shape):
    n = len(shape)
    return pl.BlockSpec(shape, lambda i: (0,) * n)


def kernel(x, edge_index, edge_attr, conv_params, lin_params):
    src = edge_index[0]
    dst = edge_index[1]

    pad = E_PAD - E
    spread = (jnp.arange(pad, dtype=jnp.int32) * 1021) % N
    src_p = jnp.concatenate([src, spread])
    dst_p = jnp.concatenate([dst, spread])
    w_p = jnp.concatenate([edge_attr, jnp.zeros((pad,), jnp.float32)])

    grid = (_ceil_div(N, BLK),)
    gspecs = {}

    src2 = src_p.reshape(E_PAD // B, B)
    dst2 = dst_p.reshape(E_PAD // B, B)

    def sc_call(g_blocked, G):
        return _make_sc_segsum(G)(g_blocked, src2, dst2, w_p)

    g0 = _pad_cols(x, LANES)[None]
    agg0 = sc_call(g0, 1)

    (wr0, wn0, b0) = conv_params[0]
    (wr1w, wn1w, b1w) = conv_params[1]
    G1 = _ceil_div(DOUTS[1], LANES)
    g1, r1 = pl.pallas_call(
        _tc_first_body,
        grid=grid,
        in_specs=[
            pl.BlockSpec((BLK, 5), lambda i: (i, 0)),
            pl.BlockSpec((NC, 1, BLK, LANES), lambda i: (0, 0, i, 0)),
            _full_spec(wr0.shape), _full_spec(wn0.shape), _full_spec(b0.shape),
            _full_spec(wn1w.shape), _full_spec(wr1w.shape), _full_spec(b1w.shape),
        ],
        out_specs=[
            pl.BlockSpec((G1, BLK, LANES), lambda i: (0, i, 0)),
            pl.BlockSpec((BLK, DOUTS[1]), lambda i: (i, 0)),
        ],
        out_shape=[
            jax.ShapeDtypeStruct((G1, N, LANES), jnp.float32),
            jax.ShapeDtypeStruct((N, DOUTS[1]), jnp.float32),
        ],
    )(x, agg0, wr0, wn0, b0, wn1w, wr1w, b1w)

    g_cur, r_cur = g1, r1
    for k in range(1, 9):
        Gk = _ceil_div(DOUTS[k], LANES)
        agg = sc_call(g_cur, Gk)
        dnext = DOUTS[k + 1]
        Gn = _ceil_div(dnext, LANES)
        (wrn, wnn, bn) = conv_params[k + 1]
        body = functools.partial(_tc_mid_body, gp=Gk, dp=DOUTS[k])
        g_cur, r_cur = pl.pallas_call(
            body,
            grid=grid,
            in_specs=[
                pl.BlockSpec((BLK, DOUTS[k]), lambda i: (i, 0)),
                pl.BlockSpec((NC, Gk, BLK, LANES), lambda i: (0, 0, i, 0)),
                _full_spec(wnn.shape), _full_spec(wrn.shape), _full_spec(bn.shape),
            ],
            out_specs=[
                pl.BlockSpec((Gn, BLK, LANES), lambda i: (0, i, 0)),
                pl.BlockSpec((BLK, dnext), lambda i: (i, 0)),
            ],
            out_shape=[
                jax.ShapeDtypeStruct((Gn, N, LANES), jnp.float32),
                jax.ShapeDtypeStruct((N, dnext), jnp.float32),
            ],
        )(r_cur, agg, wnn, wrn, bn)

    agg9 = sc_call(g_cur, 1)
    h10 = pl.pallas_call(
        _tc_last_body,
        grid=grid,
        in_specs=[
            pl.BlockSpec((BLK, 2), lambda i: (i, 0)),
            pl.BlockSpec((NC, 1, BLK, LANES), lambda i: (0, 0, i, 0)),
        ],
        out_specs=pl.BlockSpec((BLK, 2), lambda i: (i, 0)),
        out_shape=jax.ShapeDtypeStruct((N, 2), jnp.float32),
    )(r_cur, agg9)

    h = h10.reshape(-1, 396)
    (w1, b1), (w2, b2), (w3, b3) = lin_params
    h = pl.pallas_call(
        _mlp_body,
        out_shape=jax.ShapeDtypeStruct((505, 396), jnp.float32),
    )(h, w1, b1, w2, b2, w3, b3)
    h = h.reshape(-1, 2)
    sblk = 8192
    out = pl.pallas_call(
        _softmax_body,
        grid=(_ceil_div(N, sblk),),
        in_specs=[pl.BlockSpec((sblk, 2), lambda i: (i, 0))],
        out_specs=pl.BlockSpec((sblk, 2), lambda i: (i, 0)),
        out_shape=jax.ShapeDtypeStruct((N, 2), jnp.float32),
    )(h)
    return out
